# Initial kernel scaffold; baseline (speedup 1.0000x reference)
import jax
import jax.numpy as jnp
from jax.experimental import pallas as pl


def kernel(x, cu_seqlens):
    raise NotImplementedError("write your pallas kernel here")


# speedup vs baseline: 2.9487x; 2.9487x over previous
import functools

import jax
import jax.numpy as jnp
from jax import lax
from jax.experimental import pallas as pl
from jax.experimental.pallas import tpu as pltpu
from jax.experimental.pallas import tpu_sc as plsc

N_ROWS = 32768
D = 1024
B_SEGS = 16
NC = 2
NS = 16
NW = NC * NS
ROWS_PER_W = N_ROWS // NW
CHUNK = 64
N_CHUNKS = ROWS_PER_W // CHUNK
LANES = 16
GROUPS = 16
PASS_COLS = GROUPS * LANES
N_PASSES = D // PASS_COLS


def _sc_partial_sums(x, cu):
    mesh = plsc.VectorSubcoreMesh(core_axis_name="c", subcore_axis_name="s")

    @functools.partial(
        pl.kernel,
        out_type=jax.ShapeDtypeStruct((NW, B_SEGS, D), jnp.float32),
        mesh=mesh,
        scratch_types=[
            pltpu.VMEM((CHUNK, D), jnp.float32),
            pltpu.VMEM((B_SEGS, D), jnp.float32),
            pltpu.VMEM((32,), jnp.int32),
        ],
    )
    def k(x_hbm, cu_hbm, part_hbm, buf, acc, cu_v):
        wid = lax.axis_index("s") * NC + lax.axis_index("c")
        base = wid * ROWS_PER_W
        pltpu.sync_copy(cu_hbm, cu_v.at[pl.ds(0, 17)])

        def zero_body(d, _):
            col = d * LANES
            for s_ in range(B_SEGS):
                acc[s_, pl.ds(col, LANES)] = jnp.zeros((LANES,), jnp.float32)
            return 0

        lax.fori_loop(0, D // LANES, zero_body, 0)

        def chunk_body(c, _):
            cb = pl.multiple_of(base + c * CHUNK, CHUNK)
            pltpu.sync_copy(x_hbm.at[pl.ds(cb, CHUNK)], buf)

            def seg_body(s, _2):
                e_lo = jnp.where(s == 0, 0, cu_v[pl.ds(s, LANES)][0])
                e_hi = jnp.where(
                    s == B_SEGS - 1, N_ROWS, cu_v[pl.ds(s + 1, LANES)][0]
                )
                lo = jnp.maximum(e_lo, cb)
                hi = jnp.minimum(e_hi, cb + CHUNK)

                @pl.when(hi > lo)
                def _():
                    for p in range(N_PASSES):
                        base_col = p * PASS_COLS

                        def row_body(r, carry):
                            rl = r - cb
                            return tuple(
                                carry[j] + buf[rl, pl.ds(base_col + j * LANES, LANES)]
                                for j in range(GROUPS)
                            )

                        init = tuple(
                            jnp.zeros((LANES,), jnp.float32) for _ in range(GROUPS)
                        )
                        vs = lax.fori_loop(lo, hi, row_body, init)
                        for j in range(GROUPS):
                            colj = base_col + j * LANES
                            acc[s, pl.ds(colj, LANES)] = (
                                acc[s, pl.ds(colj, LANES)] + vs[j]
                            )

                return 0

            lax.fori_loop(0, B_SEGS, seg_body, 0)
            return 0

        lax.fori_loop(0, N_CHUNKS, chunk_body, 0)
        pltpu.sync_copy(acc, part_hbm.at[wid])

    return k(x, cu)


def _combine(cu, partials):
    def body(cu_ref, p_ref, o_ref):
        psum = jnp.sum(p_ref[...], axis=0)
        scalars = []
        for s_ in range(B_SEGS):
            e_lo = jnp.int32(0) if s_ == 0 else cu_ref[s_]
            e_hi = jnp.int32(N_ROWS) if s_ == B_SEGS - 1 else cu_ref[s_ + 1]
            cnt = jnp.maximum((e_hi - e_lo).astype(jnp.float32), 1.0)
            scalars.append(1.0 / cnt)
        recip = jnp.stack(scalars)
        o_ref[...] = psum * recip[:, None]

    return pl.pallas_call(
        body,
        out_shape=jax.ShapeDtypeStruct((B_SEGS, D), jnp.float32),
        in_specs=[
            pl.BlockSpec(memory_space=pltpu.SMEM),
            pl.BlockSpec(memory_space=pltpu.VMEM),
        ],
        out_specs=pl.BlockSpec(memory_space=pltpu.VMEM),
    )(cu, partials)


def kernel(x, cu_seqlens):
    partials = _sc_partial_sums(x, cu_seqlens)
    return _combine(cu_seqlens, partials)


# speedup vs baseline: 4.1754x; 1.4160x over previous
import functools

import jax
import jax.numpy as jnp
from jax import lax
from jax.experimental import pallas as pl
from jax.experimental.pallas import tpu as pltpu
from jax.experimental.pallas import tpu_sc as plsc

N_ROWS = 32768
D = 1024
B_SEGS = 16
NC = 2
NS = 16
NW = NC * NS
ROWS_PER_W = N_ROWS // NW
CHUNK = 32
N_CHUNKS = ROWS_PER_W // CHUNK
NBUF = 2
LANES = 16
GROUPS = 16
PASS_COLS = GROUPS * LANES
N_PASSES = D // PASS_COLS


def _sc_partial_sums(x, cu):
    mesh = plsc.VectorSubcoreMesh(core_axis_name="c", subcore_axis_name="s")

    @functools.partial(
        pl.kernel,
        out_type=jax.ShapeDtypeStruct((NW, B_SEGS, D), jnp.float32),
        mesh=mesh,
        scratch_types=[
            pltpu.VMEM((NBUF, CHUNK, D), jnp.float32),
            pltpu.VMEM((B_SEGS, D), jnp.float32),
            pltpu.VMEM((32,), jnp.int32),
            pltpu.SemaphoreType.DMA,
            pltpu.SemaphoreType.DMA,
        ],
    )
    def k(x_hbm, cu_hbm, part_hbm, buf, acc, cu_v, sem0, sem1):
        sems = (sem0, sem1)
        wid = lax.axis_index("s") * NC + lax.axis_index("c")
        base = wid * ROWS_PER_W
        pltpu.sync_copy(cu_hbm, cu_v.at[pl.ds(0, 17)])

        def zero_body(d, _):
            col = d * LANES
            for s_ in range(B_SEGS):
                acc[s_, pl.ds(col, LANES)] = jnp.zeros((LANES,), jnp.float32)
            return 0

        lax.fori_loop(0, D // LANES, zero_body, 0)

        def chunk_base(c):
            return pl.multiple_of(base + c * CHUNK, CHUNK)

        def issue(c, b):
            pltpu.async_copy(
                x_hbm.at[pl.ds(chunk_base(c), CHUNK)], buf.at[b], sems[b]
            )

        for b in range(NBUF):
            issue(b, b)

        def process(bref, cb):
            def seg_body(s, _2):
                e_lo = jnp.where(s == 0, 0, cu_v[pl.ds(s, LANES)][0])
                e_hi = jnp.where(
                    s == B_SEGS - 1, N_ROWS, cu_v[pl.ds(s + 1, LANES)][0]
                )
                lo = jnp.maximum(e_lo, cb)
                hi = jnp.minimum(e_hi, cb + CHUNK)

                @pl.when(hi > lo)
                def _():
                    for p in range(N_PASSES):
                        base_col = p * PASS_COLS

                        def row_body(r, carry):
                            rl = r - cb
                            return tuple(
                                carry[j]
                                + bref[rl, pl.ds(base_col + j * LANES, LANES)]
                                for j in range(GROUPS)
                            )

                        init = tuple(
                            jnp.zeros((LANES,), jnp.float32)
                            for _ in range(GROUPS)
                        )
                        vs = lax.fori_loop(lo, hi, row_body, init)
                        for j in range(GROUPS):
                            colj = base_col + j * LANES
                            acc[s, pl.ds(colj, LANES)] = (
                                acc[s, pl.ds(colj, LANES)] + vs[j]
                            )

                return 0

            lax.fori_loop(0, B_SEGS, seg_body, 0)

        def outer(g, _):
            for b in range(NBUF):
                c = g * NBUF + b
                cb = chunk_base(c)
                pltpu.make_async_copy(
                    x_hbm.at[pl.ds(cb, CHUNK)], buf.at[b], sems[b]
                ).wait()
                process(buf.at[b], cb)

                @pl.when(c + NBUF < N_CHUNKS)
                def _():
                    issue(c + NBUF, b)

            return 0

        lax.fori_loop(0, N_CHUNKS // NBUF, outer, 0)
        pltpu.sync_copy(acc, part_hbm.at[wid])

    return k(x, cu)


def _combine(cu, partials):
    def body(cu_ref, p_ref, o_ref):
        psum = jnp.sum(p_ref[...], axis=0)
        scalars = []
        for s_ in range(B_SEGS):
            e_lo = jnp.int32(0) if s_ == 0 else cu_ref[s_]
            e_hi = jnp.int32(N_ROWS) if s_ == B_SEGS - 1 else cu_ref[s_ + 1]
            cnt = jnp.maximum((e_hi - e_lo).astype(jnp.float32), 1.0)
            scalars.append(1.0 / cnt)
        recip = jnp.stack(scalars)
        o_ref[...] = psum * recip[:, None]

    return pl.pallas_call(
        body,
        out_shape=jax.ShapeDtypeStruct((B_SEGS, D), jnp.float32),
        in_specs=[
            pl.BlockSpec(memory_space=pltpu.SMEM),
            pl.BlockSpec(memory_space=pltpu.VMEM),
        ],
        out_specs=pl.BlockSpec(memory_space=pltpu.VMEM),
    )(cu, partials)


def kernel(x, cu_seqlens):
    partials = _sc_partial_sums(x, cu_seqlens)
    return _combine(cu_seqlens, partials)


# speedup vs baseline: 5.3667x; 1.2853x over previous
import functools

import jax
import jax.numpy as jnp
from jax import lax
from jax.experimental import pallas as pl
from jax.experimental.pallas import tpu as pltpu
from jax.experimental.pallas import tpu_sc as plsc

N_ROWS = 32768
D = 1024
B_SEGS = 16
M_SC = 16384
NC = 2
NS = 16
NW = NC * NS
ROWS_PER_W = M_SC // NW
CHUNK = 32
N_CHUNKS = ROWS_PER_W // CHUNK
NBUF = 2
LANES = 16
GROUPS = 16
PASS_COLS = GROUPS * LANES
N_PASSES = D // PASS_COLS
TC_BLOCK = 1024


def _seg_bounds(cu_ref):
    e = [jnp.int32(0)]
    for s_ in range(1, B_SEGS):
        e.append(cu_ref[s_])
    e.append(jnp.int32(N_ROWS))
    return e


def _sc_partial_sums(x, cu):
    mesh = plsc.VectorSubcoreMesh(core_axis_name="c", subcore_axis_name="s")

    @functools.partial(
        pl.kernel,
        out_type=jax.ShapeDtypeStruct((NW, B_SEGS, D), jnp.float32),
        mesh=mesh,
        scratch_types=[
            pltpu.VMEM((NBUF, CHUNK, D), jnp.float32),
            pltpu.VMEM((B_SEGS, D), jnp.float32),
            pltpu.VMEM((32,), jnp.int32),
            pltpu.SemaphoreType.DMA,
            pltpu.SemaphoreType.DMA,
        ],
    )
    def k(x_hbm, cu_hbm, part_hbm, buf, acc, cu_v, sem0, sem1):
        sems = (sem0, sem1)
        wid = lax.axis_index("s") * NC + lax.axis_index("c")
        base = wid * ROWS_PER_W
        pltpu.sync_copy(cu_hbm, cu_v.at[pl.ds(0, 17)])

        def zero_body(d, _):
            col = d * LANES
            for s_ in range(B_SEGS):
                acc[s_, pl.ds(col, LANES)] = jnp.zeros((LANES,), jnp.float32)
            return 0

        lax.fori_loop(0, D // LANES, zero_body, 0)

        def chunk_base(c):
            return pl.multiple_of(base + c * CHUNK, CHUNK)

        def issue(c, b):
            pltpu.async_copy(
                x_hbm.at[pl.ds(chunk_base(c), CHUNK)], buf.at[b], sems[b]
            )

        for b in range(NBUF):
            issue(b, b)

        def process(bref, cb):
            def seg_body(s, _2):
                e_lo = jnp.where(s == 0, 0, cu_v[pl.ds(s, LANES)][0])
                e_hi = jnp.where(
                    s == B_SEGS - 1, N_ROWS, cu_v[pl.ds(s + 1, LANES)][0]
                )
                lo = jnp.maximum(e_lo, cb)
                hi = jnp.minimum(e_hi, cb + CHUNK)

                @pl.when(hi > lo)
                def _():
                    for p in range(N_PASSES):
                        base_col = p * PASS_COLS

                        def row_body(r, carry):
                            rl = r - cb
                            return tuple(
                                carry[j]
                                + bref[rl, pl.ds(base_col + j * LANES, LANES)]
                                for j in range(GROUPS)
                            )

                        init = tuple(
                            jnp.zeros((LANES,), jnp.float32)
                            for _ in range(GROUPS)
                        )
                        vs = lax.fori_loop(lo, hi, row_body, init)
                        for j in range(GROUPS):
                            colj = base_col + j * LANES
                            acc[s, pl.ds(colj, LANES)] = (
                                acc[s, pl.ds(colj, LANES)] + vs[j]
                            )

                return 0

            lax.fori_loop(0, B_SEGS, seg_body, 0)

        def outer(g, _):
            for b in range(NBUF):
                c = g * NBUF + b
                cb = chunk_base(c)
                pltpu.make_async_copy(
                    x_hbm.at[pl.ds(cb, CHUNK)], buf.at[b], sems[b]
                ).wait()
                process(buf.at[b], cb)

                @pl.when(c + NBUF < N_CHUNKS)
                def _():
                    issue(c + NBUF, b)

            return 0

        lax.fori_loop(0, N_CHUNKS // NBUF, outer, 0)
        pltpu.sync_copy(acc, part_hbm.at[wid])

    return k(x, cu)


def _tc_partial_sums(x, cu):
    grid = (N_ROWS - M_SC) // TC_BLOCK

    def body(cu_ref, x_ref, o_ref):
        i = pl.program_id(0)

        @pl.when(i == 0)
        def _():
            o_ref[...] = jnp.zeros_like(o_ref)

        e = _seg_bounds(cu_ref)
        row0 = M_SC + i * TC_BLOCK
        rows = row0 + lax.broadcasted_iota(jnp.int32, (1, TC_BLOCK), 1)
        hot = [
            ((rows >= e[s_]) & (rows < e[s_ + 1])).astype(jnp.float32)
            for s_ in range(B_SEGS)
        ]
        onehot = jnp.concatenate(hot, axis=0)
        o_ref[...] += jnp.dot(
            onehot, x_ref[...], preferred_element_type=jnp.float32
        )

    return pl.pallas_call(
        body,
        grid=(grid,),
        out_shape=jax.ShapeDtypeStruct((B_SEGS, D), jnp.float32),
        in_specs=[
            pl.BlockSpec(memory_space=pltpu.SMEM),
            pl.BlockSpec(
                (TC_BLOCK, D), lambda i: (M_SC // TC_BLOCK + i, 0)
            ),
        ],
        out_specs=pl.BlockSpec((B_SEGS, D), lambda i: (0, 0)),
    )(cu, x)


def _combine(cu, partials, tc_partial):
    def body(cu_ref, p_ref, t_ref, o_ref):
        psum = jnp.sum(p_ref[...], axis=0) + t_ref[...]
        e = _seg_bounds(cu_ref)
        scalars = []
        for s_ in range(B_SEGS):
            cnt = jnp.maximum((e[s_ + 1] - e[s_]).astype(jnp.float32), 1.0)
            scalars.append(1.0 / cnt)
        recip = jnp.stack(scalars)
        o_ref[...] = psum * recip[:, None]

    return pl.pallas_call(
        body,
        out_shape=jax.ShapeDtypeStruct((B_SEGS, D), jnp.float32),
        in_specs=[
            pl.BlockSpec(memory_space=pltpu.SMEM),
            pl.BlockSpec(memory_space=pltpu.VMEM),
            pl.BlockSpec(memory_space=pltpu.VMEM),
        ],
        out_specs=pl.BlockSpec(memory_space=pltpu.VMEM),
    )(cu, partials, tc_partial)


def kernel(x, cu_seqlens):
    partials = _sc_partial_sums(x, cu_seqlens)
    tc_partial = _tc_partial_sums(x, cu_seqlens)
    return _combine(cu_seqlens, partials, tc_partial)


# speedup vs baseline: 5.8476x; 1.0896x over previous
import functools

import jax
import jax.numpy as jnp
from jax import lax
from jax.experimental import pallas as pl
from jax.experimental.pallas import tpu as pltpu
from jax.experimental.pallas import tpu_sc as plsc

N_ROWS = 32768
D = 1024
B_SEGS = 16
M_SC = 12288
NC = 2
NS = 16
NW = NC * NS
ROWS_PER_W = M_SC // NW
CHUNK = 32
N_CHUNKS = ROWS_PER_W // CHUNK
NBUF = 2
LANES = 16
GROUPS = 16
PASS_COLS = GROUPS * LANES
N_PASSES = D // PASS_COLS
TC_BLOCK = 1024


def _seg_bounds(cu_ref):
    e = [jnp.int32(0)]
    for s_ in range(1, B_SEGS):
        e.append(cu_ref[s_])
    e.append(jnp.int32(N_ROWS))
    return e


def _sc_partial_sums(x, cu):
    mesh = plsc.VectorSubcoreMesh(core_axis_name="c", subcore_axis_name="s")

    @functools.partial(
        pl.kernel,
        out_type=jax.ShapeDtypeStruct((NW, B_SEGS, D), jnp.float32),
        mesh=mesh,
        scratch_types=[
            pltpu.VMEM((NBUF, CHUNK, D), jnp.float32),
            pltpu.VMEM((B_SEGS, D), jnp.float32),
            pltpu.VMEM((32,), jnp.int32),
            pltpu.SemaphoreType.DMA,
            pltpu.SemaphoreType.DMA,
        ],
    )
    def k(x_hbm, cu_hbm, part_hbm, buf, acc, cu_v, sem0, sem1):
        sems = (sem0, sem1)
        wid = lax.axis_index("s") * NC + lax.axis_index("c")
        base = wid * ROWS_PER_W
        pltpu.sync_copy(cu_hbm, cu_v.at[pl.ds(0, 17)])

        def zero_body(d, _):
            col = d * LANES
            for s_ in range(B_SEGS):
                acc[s_, pl.ds(col, LANES)] = jnp.zeros((LANES,), jnp.float32)
            return 0

        lax.fori_loop(0, D // LANES, zero_body, 0)

        def chunk_base(c):
            return pl.multiple_of(base + c * CHUNK, CHUNK)

        def issue(c, b):
            pltpu.async_copy(
                x_hbm.at[pl.ds(chunk_base(c), CHUNK)], buf.at[b], sems[b]
            )

        for b in range(NBUF):
            issue(b, b)

        def process(bref, cb):
            def seg_body(s, _2):
                e_lo = jnp.where(s == 0, 0, cu_v[pl.ds(s, LANES)][0])
                e_hi = jnp.where(
                    s == B_SEGS - 1, N_ROWS, cu_v[pl.ds(s + 1, LANES)][0]
                )
                lo = jnp.maximum(e_lo, cb)
                hi = jnp.minimum(e_hi, cb + CHUNK)

                @pl.when(hi > lo)
                def _():
                    for p in range(N_PASSES):
                        base_col = p * PASS_COLS

                        def row_body(r, carry):
                            rl = r - cb
                            return tuple(
                                carry[j]
                                + bref[rl, pl.ds(base_col + j * LANES, LANES)]
                                for j in range(GROUPS)
                            )

                        init = tuple(
                            jnp.zeros((LANES,), jnp.float32)
                            for _ in range(GROUPS)
                        )
                        vs = lax.fori_loop(lo, hi, row_body, init)
                        for j in range(GROUPS):
                            colj = base_col + j * LANES
                            acc[s, pl.ds(colj, LANES)] = (
                                acc[s, pl.ds(colj, LANES)] + vs[j]
                            )

                return 0

            lax.fori_loop(0, B_SEGS, seg_body, 0)

        def outer(g, _):
            for b in range(NBUF):
                c = g * NBUF + b
                cb = chunk_base(c)
                pltpu.make_async_copy(
                    x_hbm.at[pl.ds(cb, CHUNK)], buf.at[b], sems[b]
                ).wait()
                process(buf.at[b], cb)

                @pl.when(c + NBUF < N_CHUNKS)
                def _():
                    issue(c + NBUF, b)

            return 0

        lax.fori_loop(0, N_CHUNKS // NBUF, outer, 0)
        pltpu.sync_copy(acc, part_hbm.at[wid])

    return k(x, cu)


def _tc_partial_sums(x, cu):
    grid = (N_ROWS - M_SC) // TC_BLOCK

    def body(cu_ref, x_ref, o_ref):
        i = pl.program_id(0)

        @pl.when(i == 0)
        def _():
            o_ref[...] = jnp.zeros_like(o_ref)

        e = _seg_bounds(cu_ref)
        row0 = M_SC + i * TC_BLOCK
        rows = row0 + lax.broadcasted_iota(jnp.int32, (1, TC_BLOCK), 1)
        hot = [
            ((rows >= e[s_]) & (rows < e[s_ + 1])).astype(jnp.float32)
            for s_ in range(B_SEGS)
        ]
        onehot = jnp.concatenate(hot, axis=0)
        o_ref[...] += jnp.dot(
            onehot, x_ref[...], preferred_element_type=jnp.float32
        )

    return pl.pallas_call(
        body,
        grid=(grid,),
        out_shape=jax.ShapeDtypeStruct((B_SEGS, D), jnp.float32),
        in_specs=[
            pl.BlockSpec(memory_space=pltpu.SMEM),
            pl.BlockSpec(
                (TC_BLOCK, D), lambda i: (M_SC // TC_BLOCK + i, 0)
            ),
        ],
        out_specs=pl.BlockSpec((B_SEGS, D), lambda i: (0, 0)),
    )(cu, x)


def _combine(cu, partials, tc_partial):
    def body(cu_ref, p_ref, t_ref, o_ref):
        psum = jnp.sum(p_ref[...], axis=0) + t_ref[...]
        e = _seg_bounds(cu_ref)
        scalars = []
        for s_ in range(B_SEGS):
            cnt = jnp.maximum((e[s_ + 1] - e[s_]).astype(jnp.float32), 1.0)
            scalars.append(1.0 / cnt)
        recip = jnp.stack(scalars)
        o_ref[...] = psum * recip[:, None]

    return pl.pallas_call(
        body,
        out_shape=jax.ShapeDtypeStruct((B_SEGS, D), jnp.float32),
        in_specs=[
            pl.BlockSpec(memory_space=pltpu.SMEM),
            pl.BlockSpec(memory_space=pltpu.VMEM),
            pl.BlockSpec(memory_space=pltpu.VMEM),
        ],
        out_specs=pl.BlockSpec(memory_space=pltpu.VMEM),
    )(cu, partials, tc_partial)


def kernel(x, cu_seqlens):
    partials = _sc_partial_sums(x, cu_seqlens)
    tc_partial = _tc_partial_sums(x, cu_seqlens)
    return _combine(cu_seqlens, partials, tc_partial)
